# baseline (speedup 1.0000x reference)
import functools

import jax
import jax.numpy as jnp
from jax import lax
from jax.experimental import pallas as pl
from jax.experimental.pallas import tpu as pltpu
from jax.experimental.pallas import tpu_sc as plsc

N = 10000
E = 320000
R = 8
D = 128
RN = R * N

NC = 2
NS = 16
NW = NC * NS
L = 16

EPW = E // NW
B = 96
SB = 12
G = 9
NCH = G * SB
EPT = NCH * B
E_PAD = NW * EPT
SROWS = 640
SLAST = N - SROWS * (NS - 1)

_MESH = plsc.VectorSubcoreMesh(core_axis_name="c", subcore_axis_name="s")
_SC_PARAMS = pltpu.CompilerParams(needs_layout_passes=False)


def _deg_body(comb_hbm, zeros_hbm, degp_hbm, comb_v, hist_v):
    c = lax.axis_index("c")
    s = lax.axis_index("s")
    wid = s * NC + c
    pltpu.sync_copy(zeros_hbm, hist_v)
    pltpu.sync_copy(comb_hbm.at[wid], comb_v)
    ones = jnp.ones((L,), jnp.float32)

    def it(i, carry):
        cv = comb_v[pl.ds(i * L, L)]
        plsc.addupdate_scatter(hist_v, [cv], ones)
        return carry

    lax.fori_loop(0, EPW // L, it, None)
    pltpu.sync_copy(hist_v, degp_hbm.at[wid])


_deg_kernel = functools.partial(
    pl.kernel,
    out_type=jax.ShapeDtypeStruct((NW, RN), jnp.float32),
    mesh=_MESH,
    compiler_params=_SC_PARAMS,
    scratch_types=[
        pltpu.VMEM((EPW,), jnp.int32),
        pltpu.VMEM((RN,), jnp.float32),
    ],
)(_deg_body)


def _norm_body(degp_ref, norm_ref):
    total = jnp.sum(degp_ref[...], axis=0)
    norm_ref[...] = 1.0 / jnp.maximum(total, 1.0)


def _norm_table(degp):
    return pl.pallas_call(
        _norm_body,
        out_shape=jax.ShapeDtypeStruct((RN // D, D), jnp.float32),
    )(degp.reshape(NW, RN // D, D))


def _norme_body(tab_hbm, comb_hbm, out_hbm, tab_v, comb_v, out_v):
    c = lax.axis_index("c")
    s = lax.axis_index("s")
    wid = s * NC + c
    pltpu.sync_copy(tab_hbm, tab_v)
    pltpu.sync_copy(comb_hbm.at[wid], comb_v)

    def it(i, carry):
        cv = comb_v[pl.ds(i * L, L)]
        out_v[pl.ds(i * L, L)] = plsc.load_gather(tab_v, [cv])
        return carry

    lax.fori_loop(0, EPT // L, it, None)
    pltpu.sync_copy(out_v, out_hbm.at[wid])


_norme_kernel = functools.partial(
    pl.kernel,
    out_type=jax.ShapeDtypeStruct((NW, EPT), jnp.float32),
    mesh=_MESH,
    compiler_params=_SC_PARAMS,
    scratch_types=[
        pltpu.VMEM((RN + L,), jnp.float32),
        pltpu.VMEM((EPT,), jnp.int32),
        pltpu.VMEM((EPT,), jnp.float32),
    ],
)(_norme_body)


def _msg_body(g_hbm, d_hbm, n_hbm, hw_hbm, z2_hbm, aggp_hbm,
              gq, dq, nq, rows0, rows1, rows2, agg,
              gs0, gs1, gs2, ss0, ss1, ss2):
    c = lax.axis_index("c")
    s = lax.axis_index("s")
    wid = s * NC + c
    r0 = pl.multiple_of(s * SROWS, 8)

    @pl.when(s < NS - 1)
    def _zero_main():
        pltpu.sync_copy(z2_hbm, agg.at[pl.ds(r0, SROWS)])

    @pl.when(s == NS - 1)
    def _zero_last():
        pltpu.sync_copy(z2_hbm.at[pl.ds(0, SLAST)], agg.at[pl.ds(r0, SLAST)])

    plsc.subcore_barrier()

    bufs = (rows0, rows1, rows2)
    gsems = (gs0, gs1, gs2)
    ssems = (ss0, ss1, ss2)

    def group(g, carry):
        row = wid * G + g
        pltpu.sync_copy(g_hbm.at[row], gq)
        pltpu.sync_copy(d_hbm.at[row], dq)
        pltpu.sync_copy(n_hbm.at[row], nq)

        gds = [pltpu.async_copy(hw_hbm.at[gq.at[k]], bufs[k % 3],
                                gsems[k % 3])
               for k in range(2)]
        sds = []
        for k in range(SB):
            buf = bufs[k % 3]
            gds[k].wait()
            bk = jnp.full((L,), k, jnp.int32)

            @plsc.parallel_loop(0, B, 1, unroll=8)
            def edge(j):
                nv = plsc.load_gather(nq, [bk, jnp.zeros((L,), jnp.int32) + j])
                for cc in range(D // L):
                    sl = pl.ds(cc * L, L)
                    buf[j, sl] = buf[j, sl] * nv

            sds.append(pltpu.async_copy(buf, agg.at[dq.at[k]],
                                        ssems[k % 3], add=True))
            if k >= 1:
                sds[k - 1].wait()
            if k + 2 < SB:
                gds.append(pltpu.async_copy(hw_hbm.at[gq.at[k + 2]],
                                            bufs[(k + 2) % 3],
                                            gsems[(k + 2) % 3]))
        sds[SB - 1].wait()
        return carry

    lax.fori_loop(0, G, group, None)
    plsc.subcore_barrier()

    @pl.when(s < NS - 1)
    def _dump_main():
        pltpu.sync_copy(agg.at[pl.ds(r0, SROWS)],
                        aggp_hbm.at[c, pl.ds(r0, SROWS)])

    @pl.when(s == NS - 1)
    def _dump_last():
        pltpu.sync_copy(agg.at[pl.ds(r0, SLAST)],
                        aggp_hbm.at[c, pl.ds(r0, SLAST)])


_msg_kernel = functools.partial(
    pl.kernel,
    out_type=jax.ShapeDtypeStruct((NC, N, D), jnp.float32),
    mesh=_MESH,
    compiler_params=_SC_PARAMS,
    scratch_types=[
        pltpu.VMEM((SB, B), jnp.int32),
        pltpu.VMEM((SB, B), jnp.int32),
        pltpu.VMEM((SB, B), jnp.float32),
        pltpu.VMEM((B, D), jnp.float32),
        pltpu.VMEM((B, D), jnp.float32),
        pltpu.VMEM((B, D), jnp.float32),
        pltpu.VMEM_SHARED((N, D), jnp.float32),
        pltpu.SemaphoreType.DMA,
        pltpu.SemaphoreType.DMA,
        pltpu.SemaphoreType.DMA,
        pltpu.SemaphoreType.DMA,
        pltpu.SemaphoreType.DMA,
        pltpu.SemaphoreType.DMA,
    ],
)(_msg_body)


_BN = 1000


def _hw_body(h_ref, w_ref, out_ref):
    out_ref[0] = jnp.dot(h_ref[...], w_ref[0],
                         preferred_element_type=jnp.float32)


def _hw_all(h, W):
    return pl.pallas_call(
        _hw_body,
        grid=(R, N // _BN),
        in_specs=[
            pl.BlockSpec((_BN, D), lambda r, i: (i, 0)),
            pl.BlockSpec((1, D, D), lambda r, i: (r, 0, 0)),
        ],
        out_specs=pl.BlockSpec((1, _BN, D), lambda r, i: (r, i, 0)),
        out_shape=jax.ShapeDtypeStruct((R, N, D), jnp.float32),
    )(h, W)


def _combine_body(a0_ref, a1_ref, h_ref, ws_ref, out_ref, *, relu):
    acc = a0_ref[...] + a1_ref[...] + jnp.dot(
        h_ref[...], ws_ref[...], preferred_element_type=jnp.float32)
    if relu:
        acc = jnp.maximum(acc, 0.0)
    out_ref[...] = acc


def _combine(a0, a1, h, Wself, relu):
    return pl.pallas_call(
        functools.partial(_combine_body, relu=relu),
        grid=(N // _BN,),
        in_specs=[
            pl.BlockSpec((_BN, D), lambda i: (i, 0)),
            pl.BlockSpec((_BN, D), lambda i: (i, 0)),
            pl.BlockSpec((_BN, D), lambda i: (i, 0)),
            pl.BlockSpec((D, D), lambda i: (0, 0)),
        ],
        out_specs=pl.BlockSpec((_BN, D), lambda i: (i, 0)),
        out_shape=jax.ShapeDtypeStruct((N, D), jnp.float32),
    )(a0, a1, h, Wself)


def _layer(h, W, Wself, g3, d3, n3, z2, relu):
    hw = _hw_all(h, W).reshape(RN, D)
    aggp = _msg_kernel(g3, d3, n3, hw, z2)
    return _combine(aggp[0], aggp[1], h, Wself, relu)


def kernel(x, edge_index, edge_type, W1, Wself1, W2, Wself2):
    src = edge_index[0].astype(jnp.int32)
    dst = edge_index[1].astype(jnp.int32)
    et = edge_type.astype(jnp.int32)
    comb = et * N + dst
    gidx = et * N + src

    npad = E_PAD - E
    pad_i = jnp.arange(npad, dtype=jnp.int32)
    comb_p = jnp.concatenate([comb, jnp.full((npad,), RN, jnp.int32)])
    gidx_p = jnp.concatenate([gidx, pad_i % RN])
    dst_p = jnp.concatenate([dst, pad_i % N])

    comb2 = comb.reshape(NW, EPW)
    comb2p = comb_p.reshape(NW, EPT)
    g3 = gidx_p.reshape(NW * G, SB, B)
    d3 = dst_p.reshape(NW * G, SB, B)
    zflat = jnp.zeros((RN,), jnp.float32)
    z2 = jnp.zeros((SROWS, D), jnp.float32)

    degp = _deg_kernel(comb2, zflat)
    normtab = jnp.concatenate([_norm_table(degp).reshape(RN),
                               jnp.zeros((L,), jnp.float32)])
    n3 = _norme_kernel(normtab, comb2p).reshape(NW * G, SB, B)

    h1 = _layer(x, W1, Wself1, g3, d3, n3, z2, relu=True)
    out = _layer(h1, W2, Wself2, g3, d3, n3, z2, relu=False)
    return out


# speedup vs baseline: 1.1137x; 1.1137x over previous
import functools

import jax
import jax.numpy as jnp
from jax import lax
from jax.experimental import pallas as pl
from jax.experimental.pallas import tpu as pltpu
from jax.experimental.pallas import tpu_sc as plsc

N = 10000
E = 320000
R = 8
D = 128
RN = R * N

NC = 2
NS = 16
NW = NC * NS
L = 16

EPW = E // NW
B = 96
SB = 12
G = 9
NCH = G * SB
EPT = NCH * B
E_PAD = NW * EPT
SROWS = 640
SLAST = N - SROWS * (NS - 1)

_MESH = plsc.VectorSubcoreMesh(core_axis_name="c", subcore_axis_name="s")
_SC_PARAMS = pltpu.CompilerParams(needs_layout_passes=False)


def _deg_body(comb_hbm, zeros_hbm, degp_hbm, comb_v, hist_v):
    c = lax.axis_index("c")
    s = lax.axis_index("s")
    wid = s * NC + c
    pltpu.sync_copy(zeros_hbm, hist_v)
    pltpu.sync_copy(comb_hbm.at[wid], comb_v)
    ones = jnp.ones((L,), jnp.float32)

    def it(i, carry):
        cv = comb_v[pl.ds(i * L, L)]
        plsc.addupdate_scatter(hist_v, [cv], ones)
        return carry

    lax.fori_loop(0, EPW // L, it, None)
    pltpu.sync_copy(hist_v, degp_hbm.at[wid])


_deg_kernel = functools.partial(
    pl.kernel,
    out_type=jax.ShapeDtypeStruct((NW, RN), jnp.float32),
    mesh=_MESH,
    compiler_params=_SC_PARAMS,
    scratch_types=[
        pltpu.VMEM((EPW,), jnp.int32),
        pltpu.VMEM((RN,), jnp.float32),
    ],
)(_deg_body)


def _norm_body(degp_ref, norm_ref):
    total = jnp.sum(degp_ref[...], axis=0)
    norm_ref[...] = 1.0 / jnp.maximum(total, 1.0)


def _norm_table(degp):
    return pl.pallas_call(
        _norm_body,
        out_shape=jax.ShapeDtypeStruct((RN // D, D), jnp.float32),
    )(degp.reshape(NW, RN // D, D))


def _norme_body(tab_hbm, comb_hbm, out_hbm, tab_v, comb_v, out_v):
    c = lax.axis_index("c")
    s = lax.axis_index("s")
    wid = s * NC + c
    pltpu.sync_copy(tab_hbm, tab_v)
    pltpu.sync_copy(comb_hbm.at[wid], comb_v)

    def it(i, carry):
        cv = comb_v[pl.ds(i * L, L)]
        out_v[pl.ds(i * L, L)] = plsc.load_gather(tab_v, [cv])
        return carry

    lax.fori_loop(0, EPT // L, it, None)
    pltpu.sync_copy(out_v, out_hbm.at[wid])


_norme_kernel = functools.partial(
    pl.kernel,
    out_type=jax.ShapeDtypeStruct((NW, EPT), jnp.float32),
    mesh=_MESH,
    compiler_params=_SC_PARAMS,
    scratch_types=[
        pltpu.VMEM((RN + L,), jnp.float32),
        pltpu.VMEM((EPT,), jnp.int32),
        pltpu.VMEM((EPT,), jnp.float32),
    ],
)(_norme_body)


def _msg_body(g_hbm, d_hbm, n_hbm, hw_hbm, z2_hbm, aggp_hbm,
              gq, dq, nq, rows0, rows1, rows2, agg,
              gs0, gs1, gs2, ss0, ss1, ss2):
    c = lax.axis_index("c")
    s = lax.axis_index("s")
    wid = s * NC + c
    r0 = pl.multiple_of(s * SROWS, 8)

    @pl.when(s < NS - 1)
    def _zero_main():
        pltpu.sync_copy(z2_hbm, agg.at[pl.ds(r0, SROWS)])

    @pl.when(s == NS - 1)
    def _zero_last():
        pltpu.sync_copy(z2_hbm.at[pl.ds(0, SLAST)], agg.at[pl.ds(r0, SLAST)])

    plsc.subcore_barrier()

    bufs = (rows0, rows1, rows2)
    gsems = (gs0, gs1, gs2)
    ssems = (ss0, ss1, ss2)

    def group(g, carry):
        row = wid * G + g
        pltpu.sync_copy(g_hbm.at[row], gq)
        pltpu.sync_copy(d_hbm.at[row], dq)
        pltpu.sync_copy(n_hbm.at[row], nq)

        gds = [pltpu.async_copy(hw_hbm.at[gq.at[k]], bufs[k % 3],
                                gsems[k % 3])
               for k in range(2)]
        sds = []
        for k in range(SB):
            buf = bufs[k % 3]
            gds[k].wait()
            bk = jnp.full((L,), k, jnp.int32)

            @plsc.parallel_loop(0, B, 1, unroll=4)
            def edge(j):
                nv = plsc.load_gather(nq, [bk, jnp.zeros((L,), jnp.int32) + j])
                for cc in range(D // L):
                    sl = pl.ds(cc * L, L)
                    buf[j, sl] = buf[j, sl] * nv

            sds.append(pltpu.async_copy(buf, agg.at[dq.at[k]],
                                        ssems[k % 3], add=True))
            if k >= 1:
                sds[k - 1].wait()
            if k + 2 < SB:
                gds.append(pltpu.async_copy(hw_hbm.at[gq.at[k + 2]],
                                            bufs[(k + 2) % 3],
                                            gsems[(k + 2) % 3]))
        sds[SB - 1].wait()
        return carry

    lax.fori_loop(0, G, group, None)
    plsc.subcore_barrier()

    @pl.when(s < NS - 1)
    def _dump_main():
        pltpu.sync_copy(agg.at[pl.ds(r0, SROWS)],
                        aggp_hbm.at[c, pl.ds(r0, SROWS)])

    @pl.when(s == NS - 1)
    def _dump_last():
        pltpu.sync_copy(agg.at[pl.ds(r0, SLAST)],
                        aggp_hbm.at[c, pl.ds(r0, SLAST)])


_msg_kernel = functools.partial(
    pl.kernel,
    out_type=jax.ShapeDtypeStruct((NC, N, D), jnp.float32),
    mesh=_MESH,
    compiler_params=_SC_PARAMS,
    scratch_types=[
        pltpu.VMEM((SB, B), jnp.int32),
        pltpu.VMEM((SB, B), jnp.int32),
        pltpu.VMEM((SB, B), jnp.float32),
        pltpu.VMEM((B, D), jnp.float32),
        pltpu.VMEM((B, D), jnp.float32),
        pltpu.VMEM((B, D), jnp.float32),
        pltpu.VMEM_SHARED((N, D), jnp.float32),
        pltpu.SemaphoreType.DMA,
        pltpu.SemaphoreType.DMA,
        pltpu.SemaphoreType.DMA,
        pltpu.SemaphoreType.DMA,
        pltpu.SemaphoreType.DMA,
        pltpu.SemaphoreType.DMA,
    ],
)(_msg_body)


_BN = 1000


def _hw_body(h_ref, w_ref, out_ref):
    out_ref[0] = jnp.dot(h_ref[...], w_ref[0],
                         preferred_element_type=jnp.float32)


def _hw_all(h, W):
    return pl.pallas_call(
        _hw_body,
        grid=(R, N // _BN),
        in_specs=[
            pl.BlockSpec((_BN, D), lambda r, i: (i, 0)),
            pl.BlockSpec((1, D, D), lambda r, i: (r, 0, 0)),
        ],
        out_specs=pl.BlockSpec((1, _BN, D), lambda r, i: (r, i, 0)),
        out_shape=jax.ShapeDtypeStruct((R, N, D), jnp.float32),
    )(h, W)


def _combine_body(a0_ref, a1_ref, h_ref, ws_ref, out_ref, *, relu):
    acc = a0_ref[...] + a1_ref[...] + jnp.dot(
        h_ref[...], ws_ref[...], preferred_element_type=jnp.float32)
    if relu:
        acc = jnp.maximum(acc, 0.0)
    out_ref[...] = acc


def _combine(a0, a1, h, Wself, relu):
    return pl.pallas_call(
        functools.partial(_combine_body, relu=relu),
        grid=(N // _BN,),
        in_specs=[
            pl.BlockSpec((_BN, D), lambda i: (i, 0)),
            pl.BlockSpec((_BN, D), lambda i: (i, 0)),
            pl.BlockSpec((_BN, D), lambda i: (i, 0)),
            pl.BlockSpec((D, D), lambda i: (0, 0)),
        ],
        out_specs=pl.BlockSpec((_BN, D), lambda i: (i, 0)),
        out_shape=jax.ShapeDtypeStruct((N, D), jnp.float32),
    )(a0, a1, h, Wself)


def _combhw_body(a0_ref, a1_ref, h_ref, ws_ref, w2_ref, h1_ref, hw2_ref):
    h1 = a0_ref[...] + a1_ref[...] + jnp.dot(
        h_ref[...], ws_ref[...], preferred_element_type=jnp.float32)
    h1 = jnp.maximum(h1, 0.0)
    h1_ref[...] = h1
    for r in range(R):
        hw2_ref[r] = jnp.dot(h1, w2_ref[r],
                             preferred_element_type=jnp.float32)


def _combine_hw(a0, a1, h, Wself, W2):
    return pl.pallas_call(
        _combhw_body,
        grid=(N // _BN,),
        in_specs=[
            pl.BlockSpec((_BN, D), lambda i: (i, 0)),
            pl.BlockSpec((_BN, D), lambda i: (i, 0)),
            pl.BlockSpec((_BN, D), lambda i: (i, 0)),
            pl.BlockSpec((D, D), lambda i: (0, 0)),
            pl.BlockSpec((R, D, D), lambda i: (0, 0, 0)),
        ],
        out_specs=[
            pl.BlockSpec((_BN, D), lambda i: (i, 0)),
            pl.BlockSpec((R, _BN, D), lambda i: (0, i, 0)),
        ],
        out_shape=[
            jax.ShapeDtypeStruct((N, D), jnp.float32),
            jax.ShapeDtypeStruct((R, N, D), jnp.float32),
        ],
    )(a0, a1, h, Wself, W2)


def kernel(x, edge_index, edge_type, W1, Wself1, W2, Wself2):
    src = edge_index[0].astype(jnp.int32)
    dst = edge_index[1].astype(jnp.int32)
    et = edge_type.astype(jnp.int32)
    comb = et * N + dst
    gidx = et * N + src

    npad = E_PAD - E
    pad_i = jnp.arange(npad, dtype=jnp.int32)
    comb_p = jnp.concatenate([comb, jnp.full((npad,), RN, jnp.int32)])
    gidx_p = jnp.concatenate([gidx, pad_i % RN])
    dst_p = jnp.concatenate([dst, pad_i % N])

    comb2 = comb.reshape(NW, EPW)
    comb2p = comb_p.reshape(NW, EPT)
    g3 = gidx_p.reshape(NW * G, SB, B)
    d3 = dst_p.reshape(NW * G, SB, B)
    zflat = jnp.zeros((RN,), jnp.float32)
    z2 = jnp.zeros((SROWS, D), jnp.float32)

    degp = _deg_kernel(comb2, zflat)
    normtab = jnp.concatenate([_norm_table(degp).reshape(RN),
                               jnp.zeros((L,), jnp.float32)])
    n3 = _norme_kernel(normtab, comb2p).reshape(NW * G, SB, B)

    hw1 = _hw_all(x, W1).reshape(RN, D)
    aggp1 = _msg_kernel(g3, d3, n3, hw1, z2)
    h1, hw2 = _combine_hw(aggp1[0], aggp1[1], x, Wself1, W2)
    aggp2 = _msg_kernel(g3, d3, n3, hw2.reshape(RN, D), z2)
    out = _combine(aggp2[0], aggp2[1], h1, Wself2, relu=False)
    return out


# speedup vs baseline: 1.1631x; 1.0443x over previous
import functools

import jax
import jax.numpy as jnp
from jax import lax
from jax.experimental import pallas as pl
from jax.experimental.pallas import tpu as pltpu
from jax.experimental.pallas import tpu_sc as plsc

N = 10000
E = 320000
R = 8
D = 128
RN = R * N

NC = 2
NS = 16
NW = NC * NS
L = 16

EPW = E // NW
B = 96
SB = 12
G = 9
NCH = G * SB
EPT = NCH * B
E_PAD = NW * EPT
SROWS = 640
SLAST = N - SROWS * (NS - 1)

_MESH = plsc.VectorSubcoreMesh(core_axis_name="c", subcore_axis_name="s")
_SC_PARAMS = pltpu.CompilerParams(needs_layout_passes=False)


def _deg_body(comb_hbm, zeros_hbm, degp_hbm, comb_v, hist_v):
    c = lax.axis_index("c")
    s = lax.axis_index("s")
    wid = s * NC + c
    pltpu.sync_copy(zeros_hbm, hist_v)
    pltpu.sync_copy(comb_hbm.at[wid], comb_v)
    ones = jnp.ones((L,), jnp.float32)

    def it(i, carry):
        cv = comb_v[pl.ds(i * L, L)]
        plsc.addupdate_scatter(hist_v, [cv], ones)
        return carry

    lax.fori_loop(0, EPW // L, it, None)
    pltpu.sync_copy(hist_v, degp_hbm.at[wid])


_deg_kernel = functools.partial(
    pl.kernel,
    out_type=jax.ShapeDtypeStruct((NW, RN), jnp.float32),
    mesh=_MESH,
    compiler_params=_SC_PARAMS,
    scratch_types=[
        pltpu.VMEM((EPW,), jnp.int32),
        pltpu.VMEM((RN,), jnp.float32),
    ],
)(_deg_body)


def _norm_body(degp_ref, norm_ref):
    total = jnp.sum(degp_ref[...], axis=0)
    norm_ref[...] = 1.0 / jnp.maximum(total, 1.0)


def _norm_table(degp):
    return pl.pallas_call(
        _norm_body,
        out_shape=jax.ShapeDtypeStruct((RN // D, D), jnp.float32),
    )(degp.reshape(NW, RN // D, D))


def _norme_body(tab_hbm, comb_hbm, out_hbm, tab_v, comb_v, out_v):
    c = lax.axis_index("c")
    s = lax.axis_index("s")
    wid = s * NC + c
    pltpu.sync_copy(tab_hbm, tab_v)
    pltpu.sync_copy(comb_hbm.at[wid], comb_v)

    def it(i, carry):
        cv = comb_v[pl.ds(i * L, L)]
        out_v[pl.ds(i * L, L)] = plsc.load_gather(tab_v, [cv])
        return carry

    lax.fori_loop(0, EPT // L, it, None)
    pltpu.sync_copy(out_v, out_hbm.at[wid])


_norme_kernel = functools.partial(
    pl.kernel,
    out_type=jax.ShapeDtypeStruct((NW, EPT), jnp.float32),
    mesh=_MESH,
    compiler_params=_SC_PARAMS,
    scratch_types=[
        pltpu.VMEM((RN + L,), jnp.float32),
        pltpu.VMEM((EPT,), jnp.int32),
        pltpu.VMEM((EPT,), jnp.float32),
    ],
)(_norme_body)


def _msg_body(g_hbm, d_hbm, n_hbm, hw_hbm, z2_hbm, aggp_hbm,
              gq, dq, nq, rows0, rows1, rows2, agg,
              gs0, gs1, gs2, ss0, ss1, ss2):
    c = lax.axis_index("c")
    s = lax.axis_index("s")
    wid = s * NC + c
    r0 = pl.multiple_of(s * SROWS, 8)

    @pl.when(s < NS - 1)
    def _zero_main():
        pltpu.sync_copy(z2_hbm, agg.at[pl.ds(r0, SROWS)])

    @pl.when(s == NS - 1)
    def _zero_last():
        pltpu.sync_copy(z2_hbm.at[pl.ds(0, SLAST)], agg.at[pl.ds(r0, SLAST)])

    plsc.subcore_barrier()

    bufs = (rows0, rows1, rows2)
    gsems = (gs0, gs1, gs2)
    ssems = (ss0, ss1, ss2)

    def group(g, carry):
        row = wid * G + g
        st0 = pltpu.async_copy(g_hbm.at[row], gq, gs0)
        st1 = pltpu.async_copy(d_hbm.at[row], dq, gs1)
        st2 = pltpu.async_copy(n_hbm.at[row], nq, gs2)
        st0.wait()
        st1.wait()
        st2.wait()

        gds = [pltpu.async_copy(hw_hbm.at[gq.at[k]], bufs[k % 3],
                                gsems[k % 3])
               for k in range(2)]
        sds = []
        for k in range(SB):
            buf = bufs[k % 3]
            gds[k].wait()
            bk = jnp.full((L,), k, jnp.int32)

            @plsc.parallel_loop(0, B, 1, unroll=4)
            def edge(j):
                nv = plsc.load_gather(nq, [bk, jnp.zeros((L,), jnp.int32) + j])
                for cc in range(D // L):
                    sl = pl.ds(cc * L, L)
                    buf[j, sl] = buf[j, sl] * nv

            sds.append(pltpu.async_copy(buf, agg.at[dq.at[k]],
                                        ssems[k % 3], add=True))
            if k >= 1:
                sds[k - 1].wait()
            if k + 2 < SB:
                gds.append(pltpu.async_copy(hw_hbm.at[gq.at[k + 2]],
                                            bufs[(k + 2) % 3],
                                            gsems[(k + 2) % 3]))
        sds[SB - 1].wait()
        return carry

    lax.fori_loop(0, G, group, None)
    plsc.subcore_barrier()

    @pl.when(s < NS - 1)
    def _dump_main():
        pltpu.sync_copy(agg.at[pl.ds(r0, SROWS)],
                        aggp_hbm.at[c, pl.ds(r0, SROWS)])

    @pl.when(s == NS - 1)
    def _dump_last():
        pltpu.sync_copy(agg.at[pl.ds(r0, SLAST)],
                        aggp_hbm.at[c, pl.ds(r0, SLAST)])


_msg_kernel = functools.partial(
    pl.kernel,
    out_type=jax.ShapeDtypeStruct((NC, N, D), jnp.float32),
    mesh=_MESH,
    compiler_params=_SC_PARAMS,
    scratch_types=[
        pltpu.VMEM((SB, B), jnp.int32),
        pltpu.VMEM((SB, B), jnp.int32),
        pltpu.VMEM((SB, B), jnp.float32),
        pltpu.VMEM((B, D), jnp.float32),
        pltpu.VMEM((B, D), jnp.float32),
        pltpu.VMEM((B, D), jnp.float32),
        pltpu.VMEM_SHARED((N, D), jnp.float32),
        pltpu.SemaphoreType.DMA,
        pltpu.SemaphoreType.DMA,
        pltpu.SemaphoreType.DMA,
        pltpu.SemaphoreType.DMA,
        pltpu.SemaphoreType.DMA,
        pltpu.SemaphoreType.DMA,
    ],
)(_msg_body)


_BN = 1000


def _hw_body(h_ref, w_ref, out_ref):
    out_ref[0] = jnp.dot(h_ref[...], w_ref[0],
                         preferred_element_type=jnp.float32)


def _hw_all(h, W):
    return pl.pallas_call(
        _hw_body,
        grid=(R, N // _BN),
        in_specs=[
            pl.BlockSpec((_BN, D), lambda r, i: (i, 0)),
            pl.BlockSpec((1, D, D), lambda r, i: (r, 0, 0)),
        ],
        out_specs=pl.BlockSpec((1, _BN, D), lambda r, i: (r, i, 0)),
        out_shape=jax.ShapeDtypeStruct((R, N, D), jnp.float32),
    )(h, W)


def _combine_body(a0_ref, a1_ref, h_ref, ws_ref, out_ref, *, relu):
    acc = a0_ref[...] + a1_ref[...] + jnp.dot(
        h_ref[...], ws_ref[...], preferred_element_type=jnp.float32)
    if relu:
        acc = jnp.maximum(acc, 0.0)
    out_ref[...] = acc


def _combine(a0, a1, h, Wself, relu):
    return pl.pallas_call(
        functools.partial(_combine_body, relu=relu),
        grid=(N // _BN,),
        in_specs=[
            pl.BlockSpec((_BN, D), lambda i: (i, 0)),
            pl.BlockSpec((_BN, D), lambda i: (i, 0)),
            pl.BlockSpec((_BN, D), lambda i: (i, 0)),
            pl.BlockSpec((D, D), lambda i: (0, 0)),
        ],
        out_specs=pl.BlockSpec((_BN, D), lambda i: (i, 0)),
        out_shape=jax.ShapeDtypeStruct((N, D), jnp.float32),
    )(a0, a1, h, Wself)


def _combhw_body(a0_ref, a1_ref, h_ref, ws_ref, w2_ref, h1_ref, hw2_ref):
    h1 = a0_ref[...] + a1_ref[...] + jnp.dot(
        h_ref[...], ws_ref[...], preferred_element_type=jnp.float32)
    h1 = jnp.maximum(h1, 0.0)
    h1_ref[...] = h1
    for r in range(R):
        hw2_ref[r] = jnp.dot(h1, w2_ref[r],
                             preferred_element_type=jnp.float32)


def _combine_hw(a0, a1, h, Wself, W2):
    return pl.pallas_call(
        _combhw_body,
        grid=(N // _BN,),
        in_specs=[
            pl.BlockSpec((_BN, D), lambda i: (i, 0)),
            pl.BlockSpec((_BN, D), lambda i: (i, 0)),
            pl.BlockSpec((_BN, D), lambda i: (i, 0)),
            pl.BlockSpec((D, D), lambda i: (0, 0)),
            pl.BlockSpec((R, D, D), lambda i: (0, 0, 0)),
        ],
        out_specs=[
            pl.BlockSpec((_BN, D), lambda i: (i, 0)),
            pl.BlockSpec((R, _BN, D), lambda i: (0, i, 0)),
        ],
        out_shape=[
            jax.ShapeDtypeStruct((N, D), jnp.float32),
            jax.ShapeDtypeStruct((R, N, D), jnp.float32),
        ],
    )(a0, a1, h, Wself, W2)


def kernel(x, edge_index, edge_type, W1, Wself1, W2, Wself2):
    src = edge_index[0].astype(jnp.int32)
    dst = edge_index[1].astype(jnp.int32)
    et = edge_type.astype(jnp.int32)
    comb = et * N + dst
    gidx = et * N + src

    npad = E_PAD - E
    pad_i = jnp.arange(npad, dtype=jnp.int32)
    comb_p = jnp.concatenate([comb, jnp.full((npad,), RN, jnp.int32)])
    gidx_p = jnp.concatenate([gidx, pad_i % RN])
    dst_p = jnp.concatenate([dst, pad_i % N])

    comb2 = comb.reshape(NW, EPW)
    comb2p = comb_p.reshape(NW, EPT)
    g3 = gidx_p.reshape(NW * G, SB, B)
    d3 = dst_p.reshape(NW * G, SB, B)
    zflat = jnp.zeros((RN,), jnp.float32)
    z2 = jnp.zeros((SROWS, D), jnp.float32)

    degp = _deg_kernel(comb2, zflat)
    normtab = jnp.concatenate([_norm_table(degp).reshape(RN),
                               jnp.zeros((L,), jnp.float32)])
    n3 = _norme_kernel(normtab, comb2p).reshape(NW * G, SB, B)

    hw1 = _hw_all(x, W1).reshape(RN, D)
    aggp1 = _msg_kernel(g3, d3, n3, hw1, z2)
    h1, hw2 = _combine_hw(aggp1[0], aggp1[1], x, Wself1, W2)
    aggp2 = _msg_kernel(g3, d3, n3, hw2.reshape(RN, D), z2)
    out = _combine(aggp2[0], aggp2[1], h1, Wself2, relu=False)
    return out
